# baseline (speedup 1.0000x reference)
import functools

import jax
import jax.numpy as jnp
from jax import lax
from jax.experimental import pallas as pl
from jax.experimental.pallas import tpu as pltpu
from jax.experimental.pallas import tpu_sc as plsc

_N_EXPERT = 100000
_D = 128
_A = 18
_B = 16384

_NC, _NS, _L = 2, 16, 16
_NW = _NC * _NS
_BPW = _B // _NW
_CHUNK = 128
_NCHUNK = _BPW // _CHUNK

_TC_BLK = 4096

_mesh = plsc.VectorSubcoreMesh(
    core_axis_name="c", subcore_axis_name="s", num_cores=_NC)


def _decode_av_body(oh_ref, av_ref):
    x = oh_ref[...]
    w = lax.broadcasted_iota(jnp.int32, (_TC_BLK, _A), 1).astype(jnp.float32)
    av_ref[...] = jnp.sum(x * w, axis=1).astype(jnp.int32)


_decode_av = pl.pallas_call(
    _decode_av_body,
    grid=((_N_EXPERT + _TC_BLK - 1) // _TC_BLK,),
    in_specs=[pl.BlockSpec((_TC_BLK, _A), lambda i: (i, 0))],
    out_specs=pl.BlockSpec((_TC_BLK,), lambda i: (i,)),
    out_shape=jax.ShapeDtypeStruct((_N_EXPERT,), jnp.int32),
)


@functools.partial(
    pl.kernel,
    mesh=_mesh,
    compiler_params=pltpu.CompilerParams(
        needs_layout_passes=False, use_tc_tiling_on_sc=False),
    out_type=(
        jax.ShapeDtypeStruct((_B, _D), jnp.float32),
        jax.ShapeDtypeStruct((_B,), jnp.int32),
    ),
    scratch_types=[
        pltpu.VMEM((_NCHUNK, _CHUNK), jnp.int32),
        pltpu.VMEM((_NCHUNK, _CHUNK), jnp.int32),
        pltpu.VMEM((_BPW, _D), jnp.float32),
        pltpu.VMEM((_BPW, _L), jnp.int32),
        pltpu.VMEM((_BPW,), jnp.int32),
        pltpu.SemaphoreType.DMA,
        pltpu.SemaphoreType.DMA,
    ],
)
def _gather_decode(states_hbm, av16_hbm, idx_hbm, out_states, out_actions,
                   idx_v, smp_v, rows_v, win_v, act_v, sem_s, sem_a):
    wid = lax.axis_index("s") * _NC + lax.axis_index("c")
    base = wid * _BPW
    pltpu.sync_copy(idx_hbm.at[pl.ds(wid * _NCHUNK, _NCHUNK)], idx_v)

    for c in range(_NCHUNK):
        for o in range(_CHUNK // _L):
            idx16 = idx_v[c, pl.ds(o * _L, _L)]
            smp_v[c, pl.ds(o * _L, _L)] = lax.shift_right_logical(idx16, 4)

    state_copies = []
    av_copies = []
    for c in range(_NCHUNK):
        state_copies.append(pltpu.async_copy(
            states_hbm.at[idx_v.at[c]],
            rows_v.at[pl.ds(c * _CHUNK, _CHUNK)], sem_s))
        av_copies.append(pltpu.async_copy(
            av16_hbm.at[smp_v.at[c]],
            win_v.at[pl.ds(c * _CHUNK, _CHUNK)], sem_a))
    for cp in av_copies:
        cp.wait()

    mask15 = jnp.full((_L,), 15, jnp.int32)
    for c in range(_NCHUNK):
        for o in range(_CHUNK // _L):
            sl = pl.ds(c * _CHUNK + o * _L, _L)
            rows16 = (c * _CHUNK + o * _L) + lax.iota(jnp.int32, _L)
            idx16 = idx_v[c, pl.ds(o * _L, _L)]
            act_v[sl] = plsc.load_gather(
                win_v, [rows16, lax.bitwise_and(idx16, mask15)])

    pltpu.sync_copy(act_v, out_actions.at[pl.ds(base, _BPW)])

    for cp in state_copies:
        cp.wait()
    pltpu.sync_copy(rows_v, out_states.at[pl.ds(base, _BPW)])


def kernel(expert_states, expert_actions, indices):
    idx2d = indices.astype(jnp.int32).reshape(_NW * _NCHUNK, _CHUNK)
    av = _decode_av(expert_actions)
    av16 = av.reshape(_N_EXPERT // _L, _L)
    states, actions = _gather_decode(expert_states, av16, idx2d)
    return (states, actions)
